# baseline (speedup 1.0000x reference)
import functools

import jax
import jax.numpy as jnp
from jax import lax
from jax.experimental import pallas as pl
from jax.experimental.pallas import tpu as pltpu

_NBUF = 24
_CCHUNKS = 3


def _tc_pool_body(x_hbm, o_ref, vmem, sem, *, ngroups, slab_rows, ccols):
    nslab = ngroups * _CCHUNKS

    def start(si):
        slot = lax.rem(si, _NBUF)
        rg = lax.div(si, _CCHUNKS)
        cc = lax.rem(si, _CCHUNKS)
        pltpu.make_async_copy(
            x_hbm.at[pl.ds(rg * slab_rows, slab_rows),
                     pl.ds(cc * ccols, ccols)],
            vmem.at[slot],
            sem.at[slot],
        ).start()

    for s in range(_NBUF - 1):
        start(s)

    def step(rg, _):
        part = jnp.zeros((slab_rows, 1), jnp.float32)
        for cc in range(_CCHUNKS):
            si = rg * _CCHUNKS + cc
            slot = lax.rem(si, _NBUF)
            nxt = si + _NBUF - 1

            @pl.when(nxt < nslab)
            def _():
                start(nxt)

            pltpu.make_async_copy(
                x_hbm.at[pl.ds(rg * slab_rows, slab_rows),
                         pl.ds(cc * ccols, ccols)],
                vmem.at[slot],
                sem.at[slot],
            ).wait()
            part = part + jnp.sum(vmem[slot], axis=1, keepdims=True)
        o_ref[pl.ds(rg * slab_rows, slab_rows)] = part
        return 0

    lax.fori_loop(0, ngroups, step, 0)


def _proj_body(p_ref, m_ref, o_ref, *, inv_n):
    s = p_ref[...] * inv_n
    o_ref[...] = jax.lax.dot_general(
        m_ref[...],
        s,
        (((1,), (0,)), ((), ())),
        preferred_element_type=jnp.float32,
    )


def kernel(x, W):
    B, C, H, Wd = x.shape
    N = H * Wd
    E = W.shape[0]
    R = B * C

    slab_rows = 8
    ngroups = R // slab_rows
    ccols = N // _CCHUNKS

    xf = x.reshape(R, N)

    rowsums = pl.pallas_call(
        functools.partial(_tc_pool_body, ngroups=ngroups,
                          slab_rows=slab_rows, ccols=ccols),
        in_specs=[pl.BlockSpec(memory_space=pl.ANY)],
        out_specs=pl.BlockSpec(memory_space=pltpu.MemorySpace.VMEM),
        out_shape=jax.ShapeDtypeStruct((R, 1), jnp.float32),
        scratch_shapes=[
            pltpu.VMEM((_NBUF, slab_rows, ccols), jnp.float32),
            pltpu.SemaphoreType.DMA((_NBUF,)),
        ],
        compiler_params=pltpu.CompilerParams(
            vmem_limit_bytes=100 * 1024 * 1024,
        ),
    )(xf)

    M = (jnp.eye(B, dtype=jnp.float32)[:, None, :, None]
         * W[None, :, None, :]).reshape(B * E, R)

    logits_flat = pl.pallas_call(
        functools.partial(_proj_body, inv_n=1.0 / N),
        in_specs=[
            pl.BlockSpec((R, 1), lambda: (0, 0)),
            pl.BlockSpec((B * E, R), lambda: (0, 0)),
        ],
        out_specs=pl.BlockSpec((B * E, 1), lambda: (0, 0)),
        out_shape=jax.ShapeDtypeStruct((B * E, 1), jnp.float32),
    )(rowsums, M)

    return logits_flat.reshape(B, E)
